# baseline (speedup 1.0000x reference)
import functools

import jax
import jax.numpy as jnp
from jax.experimental import pallas as pl
from jax.experimental.pallas import tpu as pltpu


def _graph_conv_kernel(a0_ref, a1_ref, x_ref, xi_ref, w0t_ref, w1t_ref,
                       b_ref, o_ref):
    x = x_ref[:]
    xi = xi_ref[:]
    p0 = jnp.dot(a0_ref[:], x, preferred_element_type=jnp.float32) + xi
    p1 = jnp.dot(a1_ref[:], x, preferred_element_type=jnp.float32) + xi
    o_ref[:] = (
        jnp.dot(p0, w0t_ref[:], preferred_element_type=jnp.float32)
        + jnp.dot(p1, w1t_ref[:], preferred_element_type=jnp.float32)
        + b_ref[:]
    )


@functools.partial(jax.jit, static_argnames=("bn",))
def _graph_conv(x0, a0, a1, w0t, w1t, b2d, bn):
    n, d = x0.shape
    out = w0t.shape[1]
    return pl.pallas_call(
        _graph_conv_kernel,
        grid=(n // bn,),
        in_specs=[
            pl.BlockSpec((bn, n), lambda i: (i, 0)),
            pl.BlockSpec((bn, n), lambda i: (i, 0)),
            pl.BlockSpec((n, d), lambda i: (0, 0)),
            pl.BlockSpec((bn, d), lambda i: (i, 0)),
            pl.BlockSpec((d, out), lambda i: (0, 0)),
            pl.BlockSpec((d, out), lambda i: (0, 0)),
            pl.BlockSpec((1, out), lambda i: (0, 0)),
        ],
        out_specs=pl.BlockSpec((bn, out), lambda i: (i, 0)),
        out_shape=jax.ShapeDtypeStruct((n, out), jnp.float32),
        compiler_params=pltpu.CompilerParams(
            dimension_semantics=("parallel",),
        ),
    )(a0, a1, x0, x0, w0t, w1t, b2d)


def kernel(inputs, supports, W, b):
    bsz, n, d = inputs.shape
    s = supports.shape[0]
    out_dim = W.shape[0]
    x0 = jnp.transpose(inputs, (1, 2, 0)).reshape(n, d * bsz)
    w0t = jnp.transpose(W[:, 0::s])
    w1t = jnp.transpose(W[:, 1::s])
    b2d = b.reshape(1, out_dim)

    bn = 200
    if n % bn:
        bn = n
    res = _graph_conv(x0, supports[0], supports[1], w0t, w1t, b2d, bn)
    return res.reshape(bsz, n, out_dim)
